# baseline (speedup 1.0000x reference)
import jax
import jax.numpy as jnp
from jax.experimental import pallas as pl
from jax.experimental.pallas import tpu as pltpu

_IN = 11
_HID = 32
_OUT = 3
_TBL = 262144


def _mlp_t_kernel(xt_ref, p_ref, o_ref):
    xt = xt_ref[...]
    w1 = p_ref[0:_IN, 0:_HID]
    b1 = jnp.transpose(p_ref[16:17, 0:_HID])
    w2 = p_ref[24:24 + _HID, 0:_OUT]
    b2 = jnp.transpose(p_ref[152:153, 0:_OUT])
    hT = jax.lax.dot_general(w1.astype(jnp.bfloat16), xt.astype(jnp.bfloat16),
                             (((0,), (0,)), ((), ())),
                             preferred_element_type=jnp.float32)
    hT = jnp.maximum(hT + jnp.broadcast_to(b1, hT.shape), 0.0)
    yT = jax.lax.dot_general(w2, hT, (((0,), (0,)), ((), ())),
                             preferred_element_type=jnp.float32)
    o_ref[...] = yT + jnp.broadcast_to(b2, yT.shape)


def kernel(x, slab):
    B = x.shape[0]
    xT = x.T
    n_steps = B // _TBL
    yT = pl.pallas_call(
        _mlp_t_kernel,
        out_shape=jax.ShapeDtypeStruct((_OUT, B), jnp.float32),
        grid=(n_steps,),
        in_specs=[
            pl.BlockSpec((_IN, _TBL), lambda i: (0, i)),
            pl.BlockSpec(slab.shape, lambda i: (0, 0)),
        ],
        out_specs=pl.BlockSpec((_OUT, _TBL), lambda i: (0, i)),
        compiler_params=pltpu.CompilerParams(
            dimension_semantics=("parallel",)),
    )(xT, slab)
    return yT.T


# speedup vs baseline: 1.0220x; 1.0220x over previous
import jax
import jax.numpy as jnp
from jax.experimental import pallas as pl
from jax.experimental.pallas import tpu as pltpu

_IN = 11
_HID = 32
_OUT = 3
_TBL = 131072


def _mlp_t_kernel(xt_ref, p_ref, o_ref):
    xt = xt_ref[...]
    w1 = p_ref[0:_IN, 0:_HID]
    b1 = jnp.transpose(p_ref[16:17, 0:_HID])
    w2 = p_ref[24:24 + _HID, 0:_OUT]
    b2 = jnp.transpose(p_ref[152:153, 0:_OUT])
    hT = jax.lax.dot_general(w1, xt, (((0,), (0,)), ((), ())),
                             preferred_element_type=jnp.float32)
    hT = jnp.maximum(hT + jnp.broadcast_to(b1, hT.shape), 0.0)
    yT = jax.lax.dot_general(w2, hT, (((0,), (0,)), ((), ())),
                             preferred_element_type=jnp.float32)
    o_ref[...] = yT + jnp.broadcast_to(b2, yT.shape)


def kernel(x, slab):
    B = x.shape[0]
    Bp = B
    if B % _TBL:
        Bp = ((B + _TBL - 1) // _TBL) * _TBL
        x = jnp.pad(x, ((0, Bp - B), (0, 0)))
    xT = x.T
    n_steps = Bp // _TBL
    yT = pl.pallas_call(
        _mlp_t_kernel,
        out_shape=jax.ShapeDtypeStruct((_OUT, Bp), jnp.float32),
        grid=(n_steps,),
        in_specs=[
            pl.BlockSpec((_IN, _TBL), lambda i: (0, i)),
            pl.BlockSpec(slab.shape, lambda i: (0, 0)),
        ],
        out_specs=pl.BlockSpec((_OUT, _TBL), lambda i: (0, i)),
        compiler_params=pltpu.CompilerParams(
            dimension_semantics=("parallel",)),
    )(xT, slab)
    y = yT.T
    return y[:B] if Bp != B else y
